# Initial kernel scaffold; baseline (speedup 1.0000x reference)
import jax
import jax.numpy as jnp
from jax.experimental import pallas as pl


def kernel(x, edge_index, W1, att_src1, att_dst1, b1, W2, att_src2, att_dst2, b2):
    raise NotImplementedError("write your pallas kernel here")


# speedup vs baseline: 1.0020x; 1.0020x over previous
import jax
import jax.numpy as jnp
from jax.experimental import pallas as pl

N = 10000
HEADS = 8


def _gat_conv(x, src, dst, W, att_src, att_dst, b, n_nodes, heads, ch):
    h = (x @ W).reshape(n_nodes, heads, ch)
    a_src = jnp.sum(h * att_src[None, :, :], axis=-1)
    a_dst = jnp.sum(h * att_dst[None, :, :], axis=-1)
    e = a_src[src] + a_dst[dst]
    e = jax.nn.leaky_relu(e, negative_slope=0.2)
    m = jax.ops.segment_max(e, dst, num_segments=n_nodes)
    e = jnp.exp(e - m[dst])
    s = jax.ops.segment_sum(e, dst, num_segments=n_nodes)
    alpha = e / (s[dst] + 1e-16)
    msg = h[src] * alpha[:, :, None]
    out = jax.ops.segment_sum(msg, dst, num_segments=n_nodes)
    return out.reshape(n_nodes, heads * ch) + b[None, :]


def _log_softmax_kernel(x_ref, o_ref):
    v = x_ref[...]
    m = jnp.max(v, axis=1, keepdims=True)
    s = jnp.log(jnp.sum(jnp.exp(v - m), axis=1, keepdims=True))
    o_ref[...] = v - m - s


def kernel(x, edge_index, W1, att_src1, att_dst1, b1, W2, att_src2, att_dst2, b2):
    loop = jnp.arange(N, dtype=edge_index.dtype)
    src = jnp.concatenate([edge_index[0], loop])
    dst = jnp.concatenate([edge_index[1], loop])
    h = _gat_conv(x, src, dst, W1, att_src1, att_dst1, b1, N, HEADS, 64)
    h = jax.nn.relu(h)
    out = _gat_conv(h, src, dst, W2, att_src2, att_dst2, b2, N, HEADS, 64)
    outp = jnp.pad(out, ((0, 240), (0, 0)))
    res = pl.pallas_call(
        _log_softmax_kernel,
        out_shape=jax.ShapeDtypeStruct((10240, 512), jnp.float32),
        grid=(10,),
        in_specs=[pl.BlockSpec((1024, 512), lambda i: (i, 0))],
        out_specs=pl.BlockSpec((1024, 512), lambda i: (i, 0)),
    )(outp)
    return res[:N]


# speedup vs baseline: 8.8794x; 8.8618x over previous
import functools

import jax
import jax.numpy as jnp
from jax import lax
from jax.experimental import pallas as pl
from jax.experimental.pallas import tpu as pltpu
from jax.experimental.pallas import tpu_sc as plsc

N = 10000
NP = 10240
EP = 331776
NCHUNK = 162
CW = 128
RW = 80


def _mm_heads_body(x_ref, w_ref, o_ref):
    h = jnp.dot(x_ref[...], w_ref[...], preferred_element_type=jnp.float32)
    for k in range(8):
        o_ref[k] = h[:, k * 64:(k + 1) * 64]


def _mm_heads(xp, W, kdim):
    return pl.pallas_call(
        _mm_heads_body,
        out_shape=jax.ShapeDtypeStruct((8, NP, 64), jnp.float32),
        grid=(10,),
        in_specs=[
            pl.BlockSpec((1024, kdim), lambda i: (i, 0)),
            pl.BlockSpec((kdim, 512), lambda i: (0, 0)),
        ],
        out_specs=pl.BlockSpec((8, 1024, 64), lambda i: (0, i, 0)),
    )(xp, W)


def _tables_body(hs_ref, asrc_ref, adst_ref, as_out, ad_out, g_out):
    i = pl.program_id(0)
    cols_s, cols_d, gs, gd = [], [], [], []
    for k in range(8):
        hh = hs_ref[k]
        a_s = jnp.sum(hh * asrc_ref[k][None, :], axis=1, keepdims=True)
        a_d = jnp.sum(hh * adst_ref[k][None, :], axis=1, keepdims=True)
        cols_s.append(a_s)
        cols_d.append(a_d)
        gs.append(jnp.max(a_s))
        gd.append(jnp.max(a_d))
    zz = jnp.zeros((1024, 8), jnp.float32)
    as_out[...] = jnp.concatenate(cols_s + [zz], axis=1)
    ad_out[...] = jnp.concatenate(cols_d + [zz], axis=1)
    rid = lax.broadcasted_iota(jnp.int32, (8, 16), 0)
    lid = lax.broadcasted_iota(jnp.int32, (8, 16), 1)
    G = jnp.zeros((8, 16), jnp.float32)
    for k in range(8):
        G = jnp.where((rid == 0) & (lid == k), gs[k], G)
        G = jnp.where((rid == 1) & (lid == k), gd[k], G)

    @pl.when(i == 0)
    def _():
        g_out[...] = G

    @pl.when(i > 0)
    def _():
        g_out[...] = jnp.maximum(g_out[...], G)


def _tables(Hs, att_src, att_dst):
    return pl.pallas_call(
        _tables_body,
        grid=(10,),
        in_specs=[
            pl.BlockSpec((8, 1024, 64), lambda i: (0, i, 0)),
            pl.BlockSpec((8, 64), lambda i: (0, 0)),
            pl.BlockSpec((8, 64), lambda i: (0, 0)),
        ],
        out_specs=(
            pl.BlockSpec((1024, 16), lambda i: (i, 0)),
            pl.BlockSpec((1024, 16), lambda i: (i, 0)),
            pl.BlockSpec((8, 16), lambda i: (0, 0)),
        ),
        out_shape=(
            jax.ShapeDtypeStruct((NP, 16), jnp.float32),
            jax.ShapeDtypeStruct((NP, 16), jnp.float32),
            jax.ShapeDtypeStruct((8, 16), jnp.float32),
        ),
    )(Hs, att_src, att_dst)


def _normalize(acc_ref):
    parts = []
    for k in range(8):
        blk = acc_ref[k]
        parts.append(blk[:, 0:64] / (blk[:, 64:65] + 1e-16))
    return jnp.concatenate(parts, axis=1)


def _norm_mm_body(acc_ref, b_ref, w_ref, o_ref):
    x2 = jnp.maximum(_normalize(acc_ref) + b_ref[...], 0.0)
    h = jnp.dot(x2, w_ref[...], preferred_element_type=jnp.float32)
    for k in range(8):
        o_ref[k] = h[:, k * 64:(k + 1) * 64]


def _norm_mm(acc, b, W):
    return pl.pallas_call(
        _norm_mm_body,
        out_shape=jax.ShapeDtypeStruct((8, NP, 64), jnp.float32),
        grid=(10,),
        in_specs=[
            pl.BlockSpec((8, 1024, RW), lambda i: (0, i, 0)),
            pl.BlockSpec((1, 512), lambda i: (0, 0)),
            pl.BlockSpec((512, 512), lambda i: (0, 0)),
        ],
        out_specs=pl.BlockSpec((8, 1024, 64), lambda i: (0, i, 0)),
    )(acc, b, W)


def _final_body(acc_ref, b_ref, o_ref):
    y = _normalize(acc_ref) + b_ref[...]
    m = jnp.max(y, axis=1, keepdims=True)
    z = y - m
    lse = jnp.log(jnp.sum(jnp.exp(z), axis=1, keepdims=True))
    o_ref[...] = z - lse


def _final(acc, b):
    return pl.pallas_call(
        _final_body,
        out_shape=jax.ShapeDtypeStruct((NP, 512), jnp.float32),
        grid=(10,),
        in_specs=[
            pl.BlockSpec((8, 1024, RW), lambda i: (0, i, 0)),
            pl.BlockSpec((1, 512), lambda i: (0, 0)),
        ],
        out_specs=pl.BlockSpec((1024, 512), lambda i: (i, 0)),
    )(acc, b)


def _sc_edge_body(srcb, dstb, hsf, asf, adf, g, out,
                  srcv, dstv, srcoj, sbuf, rbuf, asb, adb, zbuf, gv,
                  accsh, sem1, sem2, sem3):
    c = lax.axis_index("c")
    s = lax.axis_index("s")
    pltpu.sync_copy(srcb.at[s], srcv)
    pltpu.sync_copy(dstb.at[s], dstv)
    pltpu.sync_copy(g, gv)

    def zb(i, carry):
        for t in range(5):
            zbuf[i, pl.ds(t * 16, 16)] = jnp.zeros((16,), jnp.float32)
        return carry
    lax.fori_loop(0, 64, zb, 0)

    lid = lax.iota(jnp.int32, 16)
    gvec = gv[0, :] + gv[1, :]
    row0 = s * 640

    for pi in range(4):
        hsel = c * 4 + pi
        hoff = hsel * NP

        for k in range(10):
            pltpu.sync_copy(zbuf, accsh.at[pl.ds(row0 + k * 64, 64), :])
        plsc.subcore_barrier()

        def chunk(j, carry):
            for t in range(8):
                srcoj[0, pl.ds(t * 16, 16)] = srcv[j, pl.ds(t * 16, 16)] + hoff
            cp1 = pltpu.async_copy(asf.at[srcv.at[j]], asb, sem1)
            cp2 = pltpu.async_copy(adf.at[dstv.at[j]], adb, sem2)
            cp3 = pltpu.async_copy(hsf.at[srcoj.at[0]], rbuf, sem3)
            cp1.wait()
            cp2.wait()
            cp3.wait()

            def edge(e, cc):
                av = asb[e, :] + adb[e, :]
                av = jnp.where(av >= 0.0, av, av * 0.2)
                wv = jnp.exp(av - gvec)
                w = jnp.sum(jnp.where(lid == hsel, wv, 0.0))
                sbuf[e, pl.ds(64, 16)] = jnp.where(lid == 0, w, 0.0)
                for t in range(4):
                    sbuf[e, pl.ds(t * 16, 16)] = rbuf[e, pl.ds(t * 16, 16)] * w
                return cc
            lax.fori_loop(0, CW, edge, 0)
            pltpu.sync_copy(sbuf, accsh.at[dstv.at[j]], add=True)
            return carry
        lax.fori_loop(0, NCHUNK, chunk, 0)
        plsc.subcore_barrier()
        pltpu.sync_copy(accsh.at[pl.ds(row0, 640)],
                        out.at[pl.ds(hoff + row0, 640)])
        plsc.subcore_barrier()


_sc_edge = functools.partial(
    pl.kernel,
    out_type=jax.ShapeDtypeStruct((8 * NP, RW), jnp.float32),
    mesh=plsc.VectorSubcoreMesh(core_axis_name="c", subcore_axis_name="s"),
    compiler_params=pltpu.CompilerParams(use_tc_tiling_on_sc=False,
                                         needs_layout_passes=False),
    scratch_types=[
        pltpu.VMEM((NCHUNK, CW), jnp.int32),
        pltpu.VMEM((NCHUNK, CW), jnp.int32),
        pltpu.VMEM((1, CW), jnp.int32),
        pltpu.VMEM((CW, RW), jnp.float32),
        pltpu.VMEM((CW, 64), jnp.float32),
        pltpu.VMEM((CW, 16), jnp.float32),
        pltpu.VMEM((CW, 16), jnp.float32),
        pltpu.VMEM((64, RW), jnp.float32),
        pltpu.VMEM((8, 16), jnp.float32),
        pltpu.VMEM_SHARED((NP, RW), jnp.float32),
        pltpu.SemaphoreType.DMA,
        pltpu.SemaphoreType.DMA,
        pltpu.SemaphoreType.DMA,
    ],
)(_sc_edge_body)


def kernel(x, edge_index, W1, att_src1, att_dst1, b1, W2, att_src2, att_dst2, b2):
    xp = jnp.pad(x.astype(jnp.float32), ((0, NP - N), (0, 0)))
    ei = edge_index.astype(jnp.int32)
    loop = jnp.arange(N, dtype=jnp.int32)
    src = jnp.concatenate([ei[0], loop])
    dst = jnp.concatenate([ei[1], loop])
    pad = EP - src.shape[0]
    src = jnp.concatenate([src, jnp.full((pad,), N, jnp.int32)])
    dst = jnp.concatenate([dst, jnp.full((pad,), N, jnp.int32)])
    srcb = src.reshape(16, NCHUNK, CW)
    dstb = dst.reshape(16, NCHUNK, CW)

    Hs1 = _mm_heads(xp, W1, 128)
    As1, Ad1, G1 = _tables(Hs1, att_src1, att_dst1)
    acc1 = _sc_edge(srcb, dstb, Hs1.reshape(8 * NP, 64), As1, Ad1, G1)
    Hs2 = _norm_mm(acc1.reshape(8, NP, RW), b1.reshape(1, 512), W2)
    As2, Ad2, G2 = _tables(Hs2, att_src2, att_dst2)
    acc2 = _sc_edge(srcb, dstb, Hs2.reshape(8 * NP, 64), As2, Ad2, G2)
    out = _final(acc2.reshape(8, NP, RW), b2.reshape(1, 512))
    return out[:N]
